# baseline (speedup 1.0000x reference)
import functools

import jax
import jax.numpy as jnp
from jax import lax
from jax.experimental import pallas as pl
from jax.experimental.pallas import tpu as pltpu
from jax.experimental.pallas import tpu_sc as plsc

N = 10000
E = 320000
D_IN = 128
HID = 8
HEADS = 8
D_OUT = 64

NC = 2
NS = 16
LANES = 16
NW = NC * NS

CEDGE = 128
NCHUNK = E // CEDGE
NPAD = 10240
ROWS_PER_TILE = NPAD // NS

_f32 = jnp.float32
_i32 = jnp.int32


_GATHER_DNUMS = lax.GatherDimensionNumbers(
    offset_dims=(), collapsed_slice_dims=(0,), start_index_map=(0,))


def _vgather(v, idx):
    return lax.gather(v, idx.reshape(LANES, 1), _GATHER_DNUMS,
                      slice_sizes=(1,),
                      mode=lax.GatherScatterMode.PROMISE_IN_BOUNDS)


def _leaky(t):
    return jnp.maximum(t, 0.2 * t)


def _tc_prep1(x_ref, w1_ref, a1s_ref, a1d_ref, h_ref, nsrc_ref, ndst_ref):
    h = jnp.dot(x_ref[...], w1_ref[...], preferred_element_type=_f32)
    sel = (lax.broadcasted_iota(_i32, (HEADS * HID, HEADS), 0) // HID
           == lax.broadcasted_iota(_i32, (HEADS * HID, HEADS), 1)).astype(_f32)
    asrc = jnp.dot(h * a1s_ref[...], sel, preferred_element_type=_f32)
    adst = jnp.dot(h * a1d_ref[...], sel, preferred_element_type=_f32)
    amax = jnp.max(asrc, axis=0, keepdims=True)
    ub = _leaky(amax + adst)
    h_ref[...] = h
    nsrc_ref[...] = jnp.concatenate([asrc, jnp.zeros_like(asrc)], axis=1)
    ndst_ref[...] = jnp.concatenate([adst, ub], axis=1)


def _sc_edges1(ei_hbm, nsrc_hbm, ndst_hbm, h_hbm,
               den_out, acc_out,
               src_v, dst_v, vs_v, vd_v, ex_v, hr_v, denom_s, acc_s, sem0, sem1):
    c = lax.axis_index("c")
    s = lax.axis_index("s")
    w = s * NC + c

    @plsc.parallel_loop(0, CEDGE, unroll=8)
    def zero_body(i):
        ex_v[0, i, :] = jnp.zeros((LANES,), _f32)
        for j in range(4):
            hr_v[0, i, pl.ds(16 * j, 16)] = jnp.zeros((LANES,), _f32)

    base_r = s * ROWS_PER_TILE
    for r in range(5):
        pltpu.sync_copy(ex_v.at[0], denom_s.at[pl.ds(base_r + r * 128, 128)])
        pltpu.sync_copy(hr_v.at[0], acc_s.at[pl.ds(base_r + r * 128, 128)])
    plsc.subcore_barrier()

    lanes = lax.broadcasted_iota(_i32, (LANES,), 0)
    idx_ub = (lanes % HEADS) + HEADS
    idx_head_base = lax.shift_right_logical(lanes, 3)
    sems = (sem0, sem1)

    niter = (NCHUNK + NW - 1) // NW

    def fire(k, b):
        ch = k * NW + w

        @pl.when(ch < NCHUNK)
        def _():
            base = ch * CEDGE
            pltpu.sync_copy(ei_hbm.at[0, pl.ds(base, CEDGE)], src_v.at[b])
            pltpu.sync_copy(ei_hbm.at[1, pl.ds(base, CEDGE)], dst_v.at[b])
            pltpu.async_copy(nsrc_hbm.at[src_v.at[b]], vs_v.at[b], sems[b])
            pltpu.async_copy(ndst_hbm.at[dst_v.at[b]], vd_v.at[b], sems[b])
            pltpu.async_copy(h_hbm.at[src_v.at[b]], hr_v.at[b], sems[b])

    def consume(k, b):
        ch = k * NW + w

        @pl.when(ch < NCHUNK)
        def _():
            pltpu.make_async_copy(nsrc_hbm.at[src_v.at[b]], vs_v.at[b], sems[b]).wait()
            pltpu.make_async_copy(ndst_hbm.at[dst_v.at[b]], vd_v.at[b], sems[b]).wait()
            pltpu.make_async_copy(h_hbm.at[src_v.at[b]], hr_v.at[b], sems[b]).wait()

            @plsc.parallel_loop(0, CEDGE, unroll=16)
            def edge_body(e):
                vs = vs_v[b, e, :]
                vd = vd_v[b, e, :]
                t = vs + vd
                el = _leaky(t)
                ub = _vgather(vd, idx_ub)
                exv = jnp.exp(el - ub)
                ex_v[b, e, :] = exv
                for j in range(4):
                    hv = hr_v[b, e, pl.ds(16 * j, 16)]
                    pat = _vgather(exv, 2 * j + idx_head_base)
                    hr_v[b, e, pl.ds(16 * j, 16)] = hv * pat

            pltpu.sync_copy(ex_v.at[b], denom_s.at[dst_v.at[b]], add=True)
            pltpu.sync_copy(hr_v.at[b], acc_s.at[dst_v.at[b]], add=True)

    fire(0, 0)

    def pair_body(p, carry):
        k0 = 2 * p
        fire(k0 + 1, 1)
        consume(k0, 0)
        fire(k0 + 2, 0)
        consume(k0 + 1, 1)
        return carry

    lax.fori_loop(0, (niter + 1) // 2, pair_body, 0)
    plsc.subcore_barrier()

    pltpu.sync_copy(denom_s.at[pl.ds(base_r, ROWS_PER_TILE)],
                    den_out.at[c, pl.ds(base_r, ROWS_PER_TILE)])
    pltpu.sync_copy(acc_s.at[pl.ds(base_r, ROWS_PER_TILE)],
                    acc_out.at[c, pl.ds(base_r, ROWS_PER_TILE)])


def _tc_mid(den_ref, acc_ref, h_ref, nsrc_ref, ndst_ref, b1_ref, w2_ref,
            a2s_ref, a2d_ref, h2ex_ref, nd2_ref):
    nsrc = nsrc_ref[...]
    ndst = ndst_ref[...]
    asrc = nsrc[:, :HEADS]
    adst = ndst[:, :HEADS]
    ub = ndst[:, HEADS:]
    ex_self = jnp.exp(_leaky(asrc + adst) - ub)
    den = (den_ref[0, :N, :HEADS] + den_ref[1, :N, :HEADS]
           + ex_self)
    sel = (lax.broadcasted_iota(_i32, (HEADS, HEADS * HID), 0)
           == lax.broadcasted_iota(_i32, (HEADS, HEADS * HID), 1) // HID
           ).astype(_f32)
    ex_self_x = jnp.dot(ex_self, sel, preferred_element_type=_f32)
    den_x = jnp.dot(den, sel, preferred_element_type=_f32)
    h = h_ref[...]
    acc = acc_ref[0, :N] + acc_ref[1, :N] + h * ex_self_x
    h1 = acc / den_x + b1_ref[...]
    h1 = jnp.where(h1 > 0, h1, jnp.exp(jnp.minimum(h1, 0.0)) - 1.0)
    h2 = jnp.dot(h1, w2_ref[...], preferred_element_type=_f32)
    as2 = jnp.dot(h2, a2s_ref[...], preferred_element_type=_f32)
    ad2 = jnp.dot(h2, a2d_ref[...], preferred_element_type=_f32)
    m2 = jnp.max(as2, axis=0, keepdims=True)
    ub2 = _leaky(m2 + ad2)
    zpad = jnp.zeros((N, 15), _f32)
    h2ex_ref[...] = jnp.concatenate([h2, as2, zpad], axis=1)
    nd2_ref[...] = jnp.concatenate([ad2, ub2, zpad[:, :14]], axis=1)


def _sc_edges2(ei_hbm, h2ex_hbm, nd2_hbm,
               den_out, acc_out,
               src_v, dst_v, hs_v, vd_v, ex_v, msg_v, denom_s, acc_s, sem0, sem1):
    c = lax.axis_index("c")
    s = lax.axis_index("s")
    w = s * NC + c

    @plsc.parallel_loop(0, CEDGE, unroll=8)
    def zero_body(i):
        ex_v[0, i, :] = jnp.zeros((LANES,), _f32)
        for j in range(4):
            msg_v[0, i, pl.ds(16 * j, 16)] = jnp.zeros((LANES,), _f32)

    base_r = s * ROWS_PER_TILE
    for r in range(5):
        pltpu.sync_copy(ex_v.at[0], denom_s.at[pl.ds(base_r + r * 128, 128)])
        pltpu.sync_copy(msg_v.at[0], acc_s.at[pl.ds(base_r + r * 128, 128)])
    plsc.subcore_barrier()

    lanes = lax.broadcasted_iota(_i32, (LANES,), 0)
    zeros16 = jnp.zeros((LANES,), _i32)
    ones16 = zeros16 + 1
    msk0 = lanes < 1
    sems = (sem0, sem1)

    niter = (NCHUNK + NW - 1) // NW

    def fire(k, b):
        ch = k * NW + w

        @pl.when(ch < NCHUNK)
        def _():
            base = ch * CEDGE
            pltpu.sync_copy(ei_hbm.at[0, pl.ds(base, CEDGE)], src_v.at[b])
            pltpu.sync_copy(ei_hbm.at[1, pl.ds(base, CEDGE)], dst_v.at[b])
            pltpu.async_copy(h2ex_hbm.at[src_v.at[b]], hs_v.at[b], sems[b])
            pltpu.async_copy(nd2_hbm.at[dst_v.at[b]], vd_v.at[b], sems[b])

    def consume(k, b):
        ch = k * NW + w

        @pl.when(ch < NCHUNK)
        def _():
            pltpu.make_async_copy(h2ex_hbm.at[src_v.at[b]], hs_v.at[b], sems[b]).wait()
            pltpu.make_async_copy(nd2_hbm.at[dst_v.at[b]], vd_v.at[b], sems[b]).wait()

            @plsc.parallel_loop(0, CEDGE, unroll=16)
            def edge_body(e):
                va = hs_v[b, e, pl.ds(64, 16)]
                vd = vd_v[b, e, :]
                t = _vgather(va, zeros16) + _vgather(vd, zeros16)
                el = _leaky(t)
                ubs = _vgather(vd, ones16)
                exs = jnp.exp(el - ubs)
                ex_v[b, e, :] = jnp.where(msk0, exs, 0.0)
                for j in range(4):
                    msg_v[b, e, pl.ds(16 * j, 16)] = hs_v[b, e, pl.ds(16 * j, 16)] * exs

            pltpu.sync_copy(ex_v.at[b], denom_s.at[dst_v.at[b]], add=True)
            pltpu.sync_copy(msg_v.at[b], acc_s.at[dst_v.at[b]], add=True)

    fire(0, 0)

    def pair_body(p, carry):
        k0 = 2 * p
        fire(k0 + 1, 1)
        consume(k0, 0)
        fire(k0 + 2, 0)
        consume(k0 + 1, 1)
        return carry

    lax.fori_loop(0, (niter + 1) // 2, pair_body, 0)
    plsc.subcore_barrier()

    pltpu.sync_copy(denom_s.at[pl.ds(base_r, ROWS_PER_TILE)],
                    den_out.at[c, pl.ds(base_r, ROWS_PER_TILE)])
    pltpu.sync_copy(acc_s.at[pl.ds(base_r, ROWS_PER_TILE)],
                    acc_out.at[c, pl.ds(base_r, ROWS_PER_TILE)])


def _tc_final(den_ref, acc_ref, h2ex_ref, nd2_ref, b2_ref, out_ref):
    h2ex = h2ex_ref[...]
    nd2 = nd2_ref[...]
    h2 = h2ex[:, :D_OUT]
    as2 = h2ex[:, D_OUT:D_OUT + 1]
    ad2 = nd2[:, :1]
    ub2 = nd2[:, 1:2]
    ex2 = jnp.exp(_leaky(as2 + ad2) - ub2)
    den = den_ref[0, :N, :1] + den_ref[1, :N, :1] + ex2
    acc = acc_ref[0, :N] + acc_ref[1, :N] + h2 * ex2
    o = acc / den + b2_ref[...]
    o = o - jnp.max(o, axis=1, keepdims=True)
    out_ref[...] = o - jnp.log(jnp.sum(jnp.exp(o), axis=1, keepdims=True))


_SC_MESH = plsc.VectorSubcoreMesh(core_axis_name="c", subcore_axis_name="s",
                                  num_cores=NC, num_subcores=NS)

_sc_layer1 = functools.partial(
    pl.kernel,
    out_type=[jax.ShapeDtypeStruct((NC, NPAD, 16), _f32),
              jax.ShapeDtypeStruct((NC, NPAD, 64), _f32)],
    mesh=_SC_MESH,
    compiler_params=pltpu.CompilerParams(use_tc_tiling_on_sc=False),
    scratch_types=[
        pltpu.VMEM((2, CEDGE), _i32),
        pltpu.VMEM((2, CEDGE), _i32),
        pltpu.VMEM((2, CEDGE, 16), _f32),
        pltpu.VMEM((2, CEDGE, 16), _f32),
        pltpu.VMEM((2, CEDGE, 16), _f32),
        pltpu.VMEM((2, CEDGE, 64), _f32),
        pltpu.VMEM_SHARED((NPAD, 16), _f32),
        pltpu.VMEM_SHARED((NPAD, 64), _f32),
        pltpu.SemaphoreType.DMA,
        pltpu.SemaphoreType.DMA,
    ],
)(_sc_edges1)

_sc_layer2 = functools.partial(
    pl.kernel,
    out_type=[jax.ShapeDtypeStruct((NC, NPAD, 16), _f32),
              jax.ShapeDtypeStruct((NC, NPAD, 64), _f32)],
    mesh=_SC_MESH,
    compiler_params=pltpu.CompilerParams(use_tc_tiling_on_sc=False),
    scratch_types=[
        pltpu.VMEM((2, CEDGE), _i32),
        pltpu.VMEM((2, CEDGE), _i32),
        pltpu.VMEM((2, CEDGE, 80), _f32),
        pltpu.VMEM((2, CEDGE, 16), _f32),
        pltpu.VMEM((2, CEDGE, 16), _f32),
        pltpu.VMEM((2, CEDGE, 64), _f32),
        pltpu.VMEM_SHARED((NPAD, 16), _f32),
        pltpu.VMEM_SHARED((NPAD, 64), _f32),
        pltpu.SemaphoreType.DMA,
        pltpu.SemaphoreType.DMA,
    ],
)(_sc_edges2)


def kernel(x, edge_index, W1, att_src1, att_dst1, b1, W2, att_src2, att_dst2, b2):
    a1s = att_src1.reshape(1, HEADS * HID)
    a1d = att_dst1.reshape(1, HEADS * HID)
    a2s = att_src2.reshape(D_OUT, 1)
    a2d = att_dst2.reshape(D_OUT, 1)

    h, nsrc, ndst = pl.pallas_call(
        _tc_prep1,
        out_shape=[jax.ShapeDtypeStruct((N, HEADS * HID), _f32),
                   jax.ShapeDtypeStruct((N, 16), _f32),
                   jax.ShapeDtypeStruct((N, 16), _f32)],
    )(x, W1, a1s, a1d)

    den1, acc1 = _sc_layer1(edge_index, nsrc, ndst, h)

    h2ex, nd2 = pl.pallas_call(
        _tc_mid,
        out_shape=[jax.ShapeDtypeStruct((N, 80), _f32),
                   jax.ShapeDtypeStruct((N, 16), _f32)],
    )(den1, acc1, h, nsrc, ndst, b1.reshape(1, HEADS * HID), W2, a2s, a2d)

    den2, acc2 = _sc_layer2(edge_index, h2ex, nd2)

    out = pl.pallas_call(
        _tc_final,
        out_shape=jax.ShapeDtypeStruct((N, D_OUT), _f32),
    )(den2, acc2, h2ex, nd2, b2.reshape(1, D_OUT))
    return out


# speedup vs baseline: 1.1051x; 1.1051x over previous
import functools

import jax
import jax.numpy as jnp
from jax import lax
from jax.experimental import pallas as pl
from jax.experimental.pallas import tpu as pltpu
from jax.experimental.pallas import tpu_sc as plsc

N = 10000
E = 320000
D_IN = 128
HID = 8
HEADS = 8
D_OUT = 64

NC = 2
NS = 16
LANES = 16
NW = NC * NS

CEDGE = 128
NCHUNK = E // CEDGE
NPAD = 10240
ROWS_PER_TILE = NPAD // NS

_f32 = jnp.float32
_i32 = jnp.int32


_GATHER_DNUMS = lax.GatherDimensionNumbers(
    offset_dims=(), collapsed_slice_dims=(0,), start_index_map=(0,))


def _vgather(v, idx):
    return lax.gather(v, idx.reshape(LANES, 1), _GATHER_DNUMS,
                      slice_sizes=(1,),
                      mode=lax.GatherScatterMode.PROMISE_IN_BOUNDS)


def _leaky(t):
    return jnp.maximum(t, 0.2 * t)


def _tc_prep1(x_ref, w1_ref, a1s_ref, a1d_ref, h_ref, nsrc_ref, ndst_ref):
    h = jnp.dot(x_ref[...], w1_ref[...], preferred_element_type=_f32)
    sel = (lax.broadcasted_iota(_i32, (HEADS * HID, HEADS), 0) // HID
           == lax.broadcasted_iota(_i32, (HEADS * HID, HEADS), 1)).astype(_f32)
    asrc = jnp.dot(h * a1s_ref[...], sel, preferred_element_type=_f32)
    adst = jnp.dot(h * a1d_ref[...], sel, preferred_element_type=_f32)
    amax = jnp.max(asrc, axis=0, keepdims=True)
    ub = _leaky(amax + adst)
    h_ref[...] = h
    nsrc_ref[...] = jnp.concatenate([asrc, jnp.zeros_like(asrc)], axis=1)
    ndst_ref[...] = jnp.concatenate([adst, ub], axis=1)


def _sc_edges1(ei_hbm, nsrc_hbm, ndst_hbm, h_hbm,
               den_out, acc_out,
               src_v, dst_v, vs_v, vd_v, ex_v, hr_v, denom_s, acc_s, sem0, sem1):
    c = lax.axis_index("c")
    s = lax.axis_index("s")
    w = s * NC + c

    @plsc.parallel_loop(0, CEDGE, unroll=8)
    def zero_body(i):
        ex_v[0, i, :] = jnp.zeros((LANES,), _f32)
        for j in range(4):
            hr_v[0, i, pl.ds(16 * j, 16)] = jnp.zeros((LANES,), _f32)

    base_r = s * ROWS_PER_TILE
    for r in range(5):
        pltpu.sync_copy(ex_v.at[0], denom_s.at[pl.ds(base_r + r * 128, 128)])
        pltpu.sync_copy(hr_v.at[0], acc_s.at[pl.ds(base_r + r * 128, 128)])
    plsc.subcore_barrier()

    lanes = lax.broadcasted_iota(_i32, (LANES,), 0)
    idx_ub = (lanes % HEADS) + HEADS
    idx_head_base = lax.shift_right_logical(lanes, 3)
    sems = (sem0, sem1)

    niter = (NCHUNK + NW - 1) // NW

    def fire(k, b):
        ch = k * NW + w

        @pl.when(ch < NCHUNK)
        def _():
            base = ch * CEDGE
            pltpu.sync_copy(ei_hbm.at[0, pl.ds(base, CEDGE)], src_v.at[b])
            pltpu.sync_copy(ei_hbm.at[1, pl.ds(base, CEDGE)], dst_v.at[b])
            pltpu.async_copy(nsrc_hbm.at[src_v.at[b]], vs_v.at[b], sems[b])
            pltpu.async_copy(ndst_hbm.at[dst_v.at[b]], vd_v.at[b], sems[b])
            pltpu.async_copy(h_hbm.at[src_v.at[b]], hr_v.at[b], sems[b])

    def consume(k, b):
        ch = k * NW + w

        @pl.when(ch < NCHUNK)
        def _():
            pltpu.make_async_copy(nsrc_hbm.at[src_v.at[b]], vs_v.at[b], sems[b]).wait()
            pltpu.make_async_copy(ndst_hbm.at[dst_v.at[b]], vd_v.at[b], sems[b]).wait()
            pltpu.make_async_copy(h_hbm.at[src_v.at[b]], hr_v.at[b], sems[b]).wait()

            @plsc.parallel_loop(0, CEDGE, unroll=4)
            def edge_body(e):
                vs = vs_v[b, e, :]
                vd = vd_v[b, e, :]
                t = vs + vd
                el = _leaky(t)
                ub = _vgather(vd, idx_ub)
                exv = jnp.exp(el - ub)
                ex_v[b, e, :] = exv
                for j in range(4):
                    hv = hr_v[b, e, pl.ds(16 * j, 16)]
                    pat = _vgather(exv, 2 * j + idx_head_base)
                    hr_v[b, e, pl.ds(16 * j, 16)] = hv * pat

            pltpu.sync_copy(ex_v.at[b], denom_s.at[dst_v.at[b]], add=True)
            pltpu.sync_copy(hr_v.at[b], acc_s.at[dst_v.at[b]], add=True)

    fire(0, 0)

    def pair_body(p, carry):
        k0 = 2 * p
        fire(k0 + 1, 1)
        consume(k0, 0)
        fire(k0 + 2, 0)
        consume(k0 + 1, 1)
        return carry

    lax.fori_loop(0, (niter + 1) // 2, pair_body, 0)
    plsc.subcore_barrier()

    pltpu.sync_copy(denom_s.at[pl.ds(base_r, ROWS_PER_TILE)],
                    den_out.at[c, pl.ds(base_r, ROWS_PER_TILE)])
    pltpu.sync_copy(acc_s.at[pl.ds(base_r, ROWS_PER_TILE)],
                    acc_out.at[c, pl.ds(base_r, ROWS_PER_TILE)])


def _tc_mid(den_ref, acc_ref, h_ref, nsrc_ref, ndst_ref, b1_ref, w2_ref,
            a2s_ref, a2d_ref, h2ex_ref, nd2_ref):
    nsrc = nsrc_ref[...]
    ndst = ndst_ref[...]
    asrc = nsrc[:, :HEADS]
    adst = ndst[:, :HEADS]
    ub = ndst[:, HEADS:]
    ex_self = jnp.exp(_leaky(asrc + adst) - ub)
    den = (den_ref[0, :N, :HEADS] + den_ref[1, :N, :HEADS]
           + ex_self)
    sel = (lax.broadcasted_iota(_i32, (HEADS, HEADS * HID), 0)
           == lax.broadcasted_iota(_i32, (HEADS, HEADS * HID), 1) // HID
           ).astype(_f32)
    ex_self_x = jnp.dot(ex_self, sel, preferred_element_type=_f32)
    den_x = jnp.dot(den, sel, preferred_element_type=_f32)
    h = h_ref[...]
    acc = acc_ref[0, :N] + acc_ref[1, :N] + h * ex_self_x
    h1 = acc / den_x + b1_ref[...]
    h1 = jnp.where(h1 > 0, h1, jnp.exp(jnp.minimum(h1, 0.0)) - 1.0)
    h2 = jnp.dot(h1, w2_ref[...], preferred_element_type=_f32)
    as2 = jnp.dot(h2, a2s_ref[...], preferred_element_type=_f32)
    ad2 = jnp.dot(h2, a2d_ref[...], preferred_element_type=_f32)
    m2 = jnp.max(as2, axis=0, keepdims=True)
    ub2 = _leaky(m2 + ad2)
    zpad = jnp.zeros((N, 15), _f32)
    h2ex_ref[...] = jnp.concatenate([h2, as2, zpad], axis=1)
    nd2_ref[...] = jnp.concatenate([ad2, ub2, zpad[:, :14]], axis=1)


def _sc_edges2(ei_hbm, h2ex_hbm, nd2_hbm,
               den_out, acc_out,
               src_v, dst_v, hs_v, vd_v, ex_v, msg_v, denom_s, acc_s, sem0, sem1):
    c = lax.axis_index("c")
    s = lax.axis_index("s")
    w = s * NC + c

    @plsc.parallel_loop(0, CEDGE, unroll=8)
    def zero_body(i):
        ex_v[0, i, :] = jnp.zeros((LANES,), _f32)
        for j in range(4):
            msg_v[0, i, pl.ds(16 * j, 16)] = jnp.zeros((LANES,), _f32)

    base_r = s * ROWS_PER_TILE
    for r in range(5):
        pltpu.sync_copy(ex_v.at[0], denom_s.at[pl.ds(base_r + r * 128, 128)])
        pltpu.sync_copy(msg_v.at[0], acc_s.at[pl.ds(base_r + r * 128, 128)])
    plsc.subcore_barrier()

    lanes = lax.broadcasted_iota(_i32, (LANES,), 0)
    zeros16 = jnp.zeros((LANES,), _i32)
    ones16 = zeros16 + 1
    msk0 = lanes < 1
    sems = (sem0, sem1)

    niter = (NCHUNK + NW - 1) // NW

    def fire(k, b):
        ch = k * NW + w

        @pl.when(ch < NCHUNK)
        def _():
            base = ch * CEDGE
            pltpu.sync_copy(ei_hbm.at[0, pl.ds(base, CEDGE)], src_v.at[b])
            pltpu.sync_copy(ei_hbm.at[1, pl.ds(base, CEDGE)], dst_v.at[b])
            pltpu.async_copy(h2ex_hbm.at[src_v.at[b]], hs_v.at[b], sems[b])
            pltpu.async_copy(nd2_hbm.at[dst_v.at[b]], vd_v.at[b], sems[b])

    def consume(k, b):
        ch = k * NW + w

        @pl.when(ch < NCHUNK)
        def _():
            pltpu.make_async_copy(h2ex_hbm.at[src_v.at[b]], hs_v.at[b], sems[b]).wait()
            pltpu.make_async_copy(nd2_hbm.at[dst_v.at[b]], vd_v.at[b], sems[b]).wait()

            @plsc.parallel_loop(0, CEDGE, unroll=4)
            def edge_body(e):
                va = hs_v[b, e, pl.ds(64, 16)]
                vd = vd_v[b, e, :]
                t = _vgather(va, zeros16) + _vgather(vd, zeros16)
                el = _leaky(t)
                ubs = _vgather(vd, ones16)
                exs = jnp.exp(el - ubs)
                ex_v[b, e, :] = jnp.where(msk0, exs, 0.0)
                for j in range(4):
                    msg_v[b, e, pl.ds(16 * j, 16)] = hs_v[b, e, pl.ds(16 * j, 16)] * exs

            pltpu.sync_copy(ex_v.at[b], denom_s.at[dst_v.at[b]], add=True)
            pltpu.sync_copy(msg_v.at[b], acc_s.at[dst_v.at[b]], add=True)

    fire(0, 0)

    def pair_body(p, carry):
        k0 = 2 * p
        fire(k0 + 1, 1)
        consume(k0, 0)
        fire(k0 + 2, 0)
        consume(k0 + 1, 1)
        return carry

    lax.fori_loop(0, (niter + 1) // 2, pair_body, 0)
    plsc.subcore_barrier()

    pltpu.sync_copy(denom_s.at[pl.ds(base_r, ROWS_PER_TILE)],
                    den_out.at[c, pl.ds(base_r, ROWS_PER_TILE)])
    pltpu.sync_copy(acc_s.at[pl.ds(base_r, ROWS_PER_TILE)],
                    acc_out.at[c, pl.ds(base_r, ROWS_PER_TILE)])


def _tc_final(den_ref, acc_ref, h2ex_ref, nd2_ref, b2_ref, out_ref):
    h2ex = h2ex_ref[...]
    nd2 = nd2_ref[...]
    h2 = h2ex[:, :D_OUT]
    as2 = h2ex[:, D_OUT:D_OUT + 1]
    ad2 = nd2[:, :1]
    ub2 = nd2[:, 1:2]
    ex2 = jnp.exp(_leaky(as2 + ad2) - ub2)
    den = den_ref[0, :N, :1] + den_ref[1, :N, :1] + ex2
    acc = acc_ref[0, :N] + acc_ref[1, :N] + h2 * ex2
    o = acc / den + b2_ref[...]
    o = o - jnp.max(o, axis=1, keepdims=True)
    out_ref[...] = o - jnp.log(jnp.sum(jnp.exp(o), axis=1, keepdims=True))


_SC_MESH = plsc.VectorSubcoreMesh(core_axis_name="c", subcore_axis_name="s",
                                  num_cores=NC, num_subcores=NS)

_sc_layer1 = functools.partial(
    pl.kernel,
    out_type=[jax.ShapeDtypeStruct((NC, NPAD, 16), _f32),
              jax.ShapeDtypeStruct((NC, NPAD, 64), _f32)],
    mesh=_SC_MESH,
    compiler_params=pltpu.CompilerParams(use_tc_tiling_on_sc=False),
    scratch_types=[
        pltpu.VMEM((2, CEDGE), _i32),
        pltpu.VMEM((2, CEDGE), _i32),
        pltpu.VMEM((2, CEDGE, 16), _f32),
        pltpu.VMEM((2, CEDGE, 16), _f32),
        pltpu.VMEM((2, CEDGE, 16), _f32),
        pltpu.VMEM((2, CEDGE, 64), _f32),
        pltpu.VMEM_SHARED((NPAD, 16), _f32),
        pltpu.VMEM_SHARED((NPAD, 64), _f32),
        pltpu.SemaphoreType.DMA,
        pltpu.SemaphoreType.DMA,
    ],
)(_sc_edges1)

_sc_layer2 = functools.partial(
    pl.kernel,
    out_type=[jax.ShapeDtypeStruct((NC, NPAD, 16), _f32),
              jax.ShapeDtypeStruct((NC, NPAD, 64), _f32)],
    mesh=_SC_MESH,
    compiler_params=pltpu.CompilerParams(use_tc_tiling_on_sc=False),
    scratch_types=[
        pltpu.VMEM((2, CEDGE), _i32),
        pltpu.VMEM((2, CEDGE), _i32),
        pltpu.VMEM((2, CEDGE, 80), _f32),
        pltpu.VMEM((2, CEDGE, 16), _f32),
        pltpu.VMEM((2, CEDGE, 16), _f32),
        pltpu.VMEM((2, CEDGE, 64), _f32),
        pltpu.VMEM_SHARED((NPAD, 16), _f32),
        pltpu.VMEM_SHARED((NPAD, 64), _f32),
        pltpu.SemaphoreType.DMA,
        pltpu.SemaphoreType.DMA,
    ],
)(_sc_edges2)


def kernel(x, edge_index, W1, att_src1, att_dst1, b1, W2, att_src2, att_dst2, b2):
    a1s = att_src1.reshape(1, HEADS * HID)
    a1d = att_dst1.reshape(1, HEADS * HID)
    a2s = att_src2.reshape(D_OUT, 1)
    a2d = att_dst2.reshape(D_OUT, 1)

    h, nsrc, ndst = pl.pallas_call(
        _tc_prep1,
        out_shape=[jax.ShapeDtypeStruct((N, HEADS * HID), _f32),
                   jax.ShapeDtypeStruct((N, 16), _f32),
                   jax.ShapeDtypeStruct((N, 16), _f32)],
    )(x, W1, a1s, a1d)

    den1, acc1 = _sc_layer1(edge_index, nsrc, ndst, h)

    h2ex, nd2 = pl.pallas_call(
        _tc_mid,
        out_shape=[jax.ShapeDtypeStruct((N, 80), _f32),
                   jax.ShapeDtypeStruct((N, 16), _f32)],
    )(den1, acc1, h, nsrc, ndst, b1.reshape(1, HEADS * HID), W2, a2s, a2d)

    den2, acc2 = _sc_layer2(edge_index, h2ex, nd2)

    out = pl.pallas_call(
        _tc_final,
        out_shape=jax.ShapeDtypeStruct((N, D_OUT), _f32),
    )(den2, acc2, h2ex, nd2, b2.reshape(1, D_OUT))
    return out


# speedup vs baseline: 1.1278x; 1.0205x over previous
import functools

import jax
import jax.numpy as jnp
from jax import lax
from jax.experimental import pallas as pl
from jax.experimental.pallas import tpu as pltpu
from jax.experimental.pallas import tpu_sc as plsc

N = 10000
E = 320000
D_IN = 128
HID = 8
HEADS = 8
D_OUT = 64

NC = 2
NS = 16
LANES = 16
NW = NC * NS

CEDGE = 128
NCHUNK = E // CEDGE
NPAD = 10240
ROWS_PER_TILE = NPAD // NS

_f32 = jnp.float32
_i32 = jnp.int32


_GATHER_DNUMS = lax.GatherDimensionNumbers(
    offset_dims=(), collapsed_slice_dims=(0,), start_index_map=(0,))


def _vgather(v, idx):
    return lax.gather(v, idx.reshape(LANES, 1), _GATHER_DNUMS,
                      slice_sizes=(1,),
                      mode=lax.GatherScatterMode.PROMISE_IN_BOUNDS)


def _leaky(t):
    return jnp.maximum(t, 0.2 * t)


def _tc_prep1(x_ref, w1_ref, a1s_ref, a1d_ref, h_ref, nsrc_ref, ndst_ref):
    h = jnp.dot(x_ref[...], w1_ref[...], preferred_element_type=_f32)
    sel = (lax.broadcasted_iota(_i32, (HEADS * HID, HEADS), 0) // HID
           == lax.broadcasted_iota(_i32, (HEADS * HID, HEADS), 1)).astype(_f32)
    asrc = jnp.dot(h * a1s_ref[...], sel, preferred_element_type=_f32)
    adst = jnp.dot(h * a1d_ref[...], sel, preferred_element_type=_f32)
    amax = jnp.max(asrc, axis=0, keepdims=True)
    ub = _leaky(amax + adst)
    h_ref[...] = h
    nsrc_ref[...] = jnp.concatenate([asrc, jnp.zeros_like(asrc)], axis=1)
    ndst_ref[...] = jnp.concatenate([adst, ub], axis=1)


def _sc_edges1(ei_hbm, nsrc_hbm, ndst_hbm, h_hbm,
               da_out,
               src_v, dst_v, vs_v, vd_v, em_v, hr_v, da_s, sem0, sem1):
    c = lax.axis_index("c")
    s = lax.axis_index("s")
    w = s * NC + c

    @plsc.parallel_loop(0, CEDGE, unroll=8)
    def zero_body(i):
        for j in range(5):
            em_v[0, i, pl.ds(16 * j, 16)] = jnp.zeros((LANES,), _f32)

    base_r = s * ROWS_PER_TILE
    for r in range(5):
        pltpu.sync_copy(em_v.at[0], da_s.at[pl.ds(base_r + r * 128, 128)])
    plsc.subcore_barrier()

    lanes = lax.broadcasted_iota(_i32, (LANES,), 0)
    idx_ub = (lanes % HEADS) + HEADS
    idx_head_base = lax.shift_right_logical(lanes, 3)
    sems = (sem0, sem1)

    niter = (NCHUNK + NW - 1) // NW

    def fire(k, b):
        ch = k * NW + w

        @pl.when(ch < NCHUNK)
        def _():
            base = ch * CEDGE
            pltpu.sync_copy(ei_hbm.at[0, pl.ds(base, CEDGE)], src_v.at[b])
            pltpu.sync_copy(ei_hbm.at[1, pl.ds(base, CEDGE)], dst_v.at[b])
            pltpu.async_copy(nsrc_hbm.at[src_v.at[b]], vs_v.at[b], sems[b])
            pltpu.async_copy(ndst_hbm.at[dst_v.at[b]], vd_v.at[b], sems[b])
            pltpu.async_copy(h_hbm.at[src_v.at[b]], hr_v.at[b], sems[b])

    def consume(k, b):
        ch = k * NW + w

        @pl.when(ch < NCHUNK)
        def _():
            pltpu.make_async_copy(nsrc_hbm.at[src_v.at[b]], vs_v.at[b], sems[b]).wait()
            pltpu.make_async_copy(ndst_hbm.at[dst_v.at[b]], vd_v.at[b], sems[b]).wait()
            pltpu.make_async_copy(h_hbm.at[src_v.at[b]], hr_v.at[b], sems[b]).wait()

            @plsc.parallel_loop(0, CEDGE, unroll=4)
            def edge_body(e):
                vs = vs_v[b, e, :]
                vd = vd_v[b, e, :]
                t = vs + vd
                el = _leaky(t)
                ub = _vgather(vd, idx_ub)
                exv = jnp.exp(el - ub)
                em_v[b, e, pl.ds(0, 16)] = exv
                for j in range(4):
                    hv = hr_v[b, e, pl.ds(16 * j, 16)]
                    pat = _vgather(exv, 2 * j + idx_head_base)
                    em_v[b, e, pl.ds(16 + 16 * j, 16)] = hv * pat

            pltpu.sync_copy(em_v.at[b], da_s.at[dst_v.at[b]], add=True)

    fire(0, 0)

    def pair_body(p, carry):
        k0 = 2 * p
        fire(k0 + 1, 1)
        consume(k0, 0)
        fire(k0 + 2, 0)
        consume(k0 + 1, 1)
        return carry

    lax.fori_loop(0, (niter + 1) // 2, pair_body, 0)
    plsc.subcore_barrier()

    pltpu.sync_copy(da_s.at[pl.ds(base_r, ROWS_PER_TILE)],
                    da_out.at[c, pl.ds(base_r, ROWS_PER_TILE)])


def _tc_mid(da_ref, h_ref, nsrc_ref, ndst_ref, b1_ref, w2_ref,
            a2s_ref, a2d_ref, h2ex_ref, nd2_ref):
    nsrc = nsrc_ref[...]
    ndst = ndst_ref[...]
    asrc = nsrc[:, :HEADS]
    adst = ndst[:, :HEADS]
    ub = ndst[:, HEADS:]
    ex_self = jnp.exp(_leaky(asrc + adst) - ub)
    den = (da_ref[0, :N, :HEADS] + da_ref[1, :N, :HEADS]
           + ex_self)
    sel = (lax.broadcasted_iota(_i32, (HEADS, HEADS * HID), 0)
           == lax.broadcasted_iota(_i32, (HEADS, HEADS * HID), 1) // HID
           ).astype(_f32)
    ex_self_x = jnp.dot(ex_self, sel, preferred_element_type=_f32)
    den_x = jnp.dot(den, sel, preferred_element_type=_f32)
    h = h_ref[...]
    acc = da_ref[0, :N, 16:] + da_ref[1, :N, 16:] + h * ex_self_x
    h1 = acc / den_x + b1_ref[...]
    h1 = jnp.where(h1 > 0, h1, jnp.exp(jnp.minimum(h1, 0.0)) - 1.0)
    h2 = jnp.dot(h1, w2_ref[...], preferred_element_type=_f32)
    as2 = jnp.dot(h2, a2s_ref[...], preferred_element_type=_f32)
    ad2 = jnp.dot(h2, a2d_ref[...], preferred_element_type=_f32)
    m2 = jnp.max(as2, axis=0, keepdims=True)
    ub2 = _leaky(m2 + ad2)
    zpad = jnp.zeros((N, 15), _f32)
    h2ex_ref[...] = jnp.concatenate([h2, as2, zpad], axis=1)
    nd2_ref[...] = jnp.concatenate([ad2, ub2, zpad[:, :14]], axis=1)


def _sc_edges2(ei_hbm, h2ex_hbm, nd2_hbm,
               da_out,
               src_v, dst_v, hs_v, vd_v, em_v, da_s, sem0, sem1):
    c = lax.axis_index("c")
    s = lax.axis_index("s")
    w = s * NC + c

    @plsc.parallel_loop(0, CEDGE, unroll=8)
    def zero_body(i):
        for j in range(5):
            em_v[0, i, pl.ds(16 * j, 16)] = jnp.zeros((LANES,), _f32)

    base_r = s * ROWS_PER_TILE
    for r in range(5):
        pltpu.sync_copy(em_v.at[0], da_s.at[pl.ds(base_r + r * 128, 128)])
    plsc.subcore_barrier()

    lanes = lax.broadcasted_iota(_i32, (LANES,), 0)
    zeros16 = jnp.zeros((LANES,), _i32)
    ones16 = zeros16 + 1
    msk0 = lanes < 1
    sems = (sem0, sem1)

    niter = (NCHUNK + NW - 1) // NW

    def fire(k, b):
        ch = k * NW + w

        @pl.when(ch < NCHUNK)
        def _():
            base = ch * CEDGE
            pltpu.sync_copy(ei_hbm.at[0, pl.ds(base, CEDGE)], src_v.at[b])
            pltpu.sync_copy(ei_hbm.at[1, pl.ds(base, CEDGE)], dst_v.at[b])
            pltpu.async_copy(h2ex_hbm.at[src_v.at[b]], hs_v.at[b], sems[b])
            pltpu.async_copy(nd2_hbm.at[dst_v.at[b]], vd_v.at[b], sems[b])

    def consume(k, b):
        ch = k * NW + w

        @pl.when(ch < NCHUNK)
        def _():
            pltpu.make_async_copy(h2ex_hbm.at[src_v.at[b]], hs_v.at[b], sems[b]).wait()
            pltpu.make_async_copy(nd2_hbm.at[dst_v.at[b]], vd_v.at[b], sems[b]).wait()

            @plsc.parallel_loop(0, CEDGE, unroll=4)
            def edge_body(e):
                va = hs_v[b, e, pl.ds(64, 16)]
                vd = vd_v[b, e, :]
                t = _vgather(va, zeros16) + _vgather(vd, zeros16)
                el = _leaky(t)
                ubs = _vgather(vd, ones16)
                exs = jnp.exp(el - ubs)
                em_v[b, e, pl.ds(0, 16)] = jnp.where(msk0, exs, 0.0)
                for j in range(4):
                    em_v[b, e, pl.ds(16 + 16 * j, 16)] = hs_v[b, e, pl.ds(16 * j, 16)] * exs

            pltpu.sync_copy(em_v.at[b], da_s.at[dst_v.at[b]], add=True)

    fire(0, 0)

    def pair_body(p, carry):
        k0 = 2 * p
        fire(k0 + 1, 1)
        consume(k0, 0)
        fire(k0 + 2, 0)
        consume(k0 + 1, 1)
        return carry

    lax.fori_loop(0, (niter + 1) // 2, pair_body, 0)
    plsc.subcore_barrier()

    pltpu.sync_copy(da_s.at[pl.ds(base_r, ROWS_PER_TILE)],
                    da_out.at[c, pl.ds(base_r, ROWS_PER_TILE)])


def _tc_final(da_ref, h2ex_ref, nd2_ref, b2_ref, out_ref):
    h2ex = h2ex_ref[...]
    nd2 = nd2_ref[...]
    h2 = h2ex[:, :D_OUT]
    as2 = h2ex[:, D_OUT:D_OUT + 1]
    ad2 = nd2[:, :1]
    ub2 = nd2[:, 1:2]
    ex2 = jnp.exp(_leaky(as2 + ad2) - ub2)
    den = da_ref[0, :N, :1] + da_ref[1, :N, :1] + ex2
    acc = da_ref[0, :N, 16:] + da_ref[1, :N, 16:] + h2 * ex2
    o = acc / den + b2_ref[...]
    o = o - jnp.max(o, axis=1, keepdims=True)
    out_ref[...] = o - jnp.log(jnp.sum(jnp.exp(o), axis=1, keepdims=True))


_SC_MESH = plsc.VectorSubcoreMesh(core_axis_name="c", subcore_axis_name="s",
                                  num_cores=NC, num_subcores=NS)

_sc_layer1 = functools.partial(
    pl.kernel,
    out_type=jax.ShapeDtypeStruct((NC, NPAD, 80), _f32),
    mesh=_SC_MESH,
    compiler_params=pltpu.CompilerParams(use_tc_tiling_on_sc=False),
    scratch_types=[
        pltpu.VMEM((2, CEDGE), _i32),
        pltpu.VMEM((2, CEDGE), _i32),
        pltpu.VMEM((2, CEDGE, 16), _f32),
        pltpu.VMEM((2, CEDGE, 16), _f32),
        pltpu.VMEM((2, CEDGE, 80), _f32),
        pltpu.VMEM((2, CEDGE, 64), _f32),
        pltpu.VMEM_SHARED((NPAD, 80), _f32),
        pltpu.SemaphoreType.DMA,
        pltpu.SemaphoreType.DMA,
    ],
)(_sc_edges1)

_sc_layer2 = functools.partial(
    pl.kernel,
    out_type=jax.ShapeDtypeStruct((NC, NPAD, 80), _f32),
    mesh=_SC_MESH,
    compiler_params=pltpu.CompilerParams(use_tc_tiling_on_sc=False),
    scratch_types=[
        pltpu.VMEM((2, CEDGE), _i32),
        pltpu.VMEM((2, CEDGE), _i32),
        pltpu.VMEM((2, CEDGE, 80), _f32),
        pltpu.VMEM((2, CEDGE, 16), _f32),
        pltpu.VMEM((2, CEDGE, 80), _f32),
        pltpu.VMEM_SHARED((NPAD, 80), _f32),
        pltpu.SemaphoreType.DMA,
        pltpu.SemaphoreType.DMA,
    ],
)(_sc_edges2)


def kernel(x, edge_index, W1, att_src1, att_dst1, b1, W2, att_src2, att_dst2, b2):
    a1s = att_src1.reshape(1, HEADS * HID)
    a1d = att_dst1.reshape(1, HEADS * HID)
    a2s = att_src2.reshape(D_OUT, 1)
    a2d = att_dst2.reshape(D_OUT, 1)

    h, nsrc, ndst = pl.pallas_call(
        _tc_prep1,
        out_shape=[jax.ShapeDtypeStruct((N, HEADS * HID), _f32),
                   jax.ShapeDtypeStruct((N, 16), _f32),
                   jax.ShapeDtypeStruct((N, 16), _f32)],
    )(x, W1, a1s, a1d)

    da1 = _sc_layer1(edge_index, nsrc, ndst, h)

    h2ex, nd2 = pl.pallas_call(
        _tc_mid,
        out_shape=[jax.ShapeDtypeStruct((N, 80), _f32),
                   jax.ShapeDtypeStruct((N, 16), _f32)],
    )(da1, h, nsrc, ndst, b1.reshape(1, HEADS * HID), W2, a2s, a2d)

    da2 = _sc_layer2(edge_index, h2ex, nd2)

    out = pl.pallas_call(
        _tc_final,
        out_shape=jax.ShapeDtypeStruct((N, D_OUT), _f32),
    )(da2, h2ex, nd2, b2.reshape(1, D_OUT))
    return out
